# baseline (speedup 1.0000x reference)
import functools

import jax
import jax.numpy as jnp
from jax import lax
from jax.experimental import pallas as pl
from jax.experimental.pallas import tpu as pltpu
from jax.experimental.pallas import tpu_sc as plsc

ENT_DIM = 64
HID_DIM = 32
NREL_PAD = 1024
NC, NS = 2, 16
CHUNK = 128
TB = 256
TBB = 1024
FPC = ENT_DIM // NC


def _sc_gather_t(ent_t, hidx3, tidx3, n_ent):
    nch = hidx3.shape[1]
    bps = nch * CHUNK
    batch = NS * bps
    mesh = plsc.VectorSubcoreMesh(
        core_axis_name="c", subcore_axis_name="s",
        num_cores=NC, num_subcores=NS)

    @functools.partial(
        pl.kernel,
        out_type=[
            jax.ShapeDtypeStruct((ENT_DIM, batch), jnp.float32),
            jax.ShapeDtypeStruct((ENT_DIM, batch), jnp.float32),
        ],
        mesh=mesh,
        scratch_types=[
            pltpu.VMEM((nch, CHUNK), jnp.int32),
            pltpu.VMEM((nch, CHUNK), jnp.int32),
            pltpu.VMEM((2, bps), jnp.float32),
            pltpu.VMEM((bps,), jnp.float32),
            pltpu.VMEM_SHARED((n_ent,), jnp.float32),
            pltpu.VMEM_SHARED((n_ent,), jnp.float32),
            pltpu.SemaphoreType.DMA,
            pltpu.SemaphoreType.DMA,
            pltpu.SemaphoreType.DMA,
        ],
    )
    def k(ent_hbm, hidx_hbm, tidx_hbm, h_out, t_out,
          hv, tv, hbuf, tbuf, row_a, row_b, rsem, gsem, osem):
        cid = lax.axis_index("c")
        sid = lax.axis_index("s")
        f0 = cid * FPC
        pltpu.sync_copy(hidx_hbm.at[sid], hv)
        pltpu.sync_copy(tidx_hbm.at[sid], tv)
        bufs = (row_a, row_b)
        base = sid * bps

        def _stage_row(f, dst):
            pltpu.async_copy(ent_hbm.at[f0 + f], dst, rsem)

        def _drain_row(f, dst):
            pltpu.make_async_copy(ent_hbm.at[f0 + f], dst, rsem).wait()

        @pl.when(sid == 0)
        def _():
            _stage_row(0, bufs[0])

        for f in range(FPC):
            cur = bufs[f % 2]

            @pl.when(sid == 0)
            def _():
                _drain_row(f, cur)

            plsc.subcore_barrier()
            if f + 1 < FPC:
                @pl.when(sid == 0)
                def _():
                    _stage_row(f + 1, bufs[(f + 1) % 2])
            pb = f % 2
            if f >= 2:
                pltpu.make_async_copy(
                    hbuf.at[pb], h_out.at[f0 + f - 2, pl.ds(base, bps)],
                    osem).wait()
            if f >= 1:
                pltpu.make_async_copy(
                    tbuf, t_out.at[f0 + f - 1, pl.ds(base, bps)],
                    osem).wait()
            copies = []
            for j in range(nch):
                copies.append(pltpu.async_copy(
                    cur.at[hv.at[j]], hbuf.at[pb, pl.ds(j * CHUNK, CHUNK)],
                    gsem))
                copies.append(pltpu.async_copy(
                    cur.at[tv.at[j]], tbuf.at[pl.ds(j * CHUNK, CHUNK)],
                    gsem))
            for c in copies:
                c.wait()
            pltpu.async_copy(
                hbuf.at[pb], h_out.at[f0 + f, pl.ds(base, bps)], osem)
            pltpu.async_copy(
                tbuf, t_out.at[f0 + f, pl.ds(base, bps)], osem)
        pltpu.make_async_copy(
            hbuf.at[(FPC - 2) % 2],
            h_out.at[f0 + FPC - 2, pl.ds(base, bps)], osem).wait()
        pltpu.make_async_copy(
            hbuf.at[(FPC - 1) % 2],
            h_out.at[f0 + FPC - 1, pl.ds(base, bps)], osem).wait()
        pltpu.make_async_copy(
            tbuf, t_out.at[f0 + FPC - 1, pl.ds(base, bps)], osem).wait()

    return k(ent_t, hidx3, tidx3)


def _tca_body(rel_ref, tr_ref, re_ref, mg_ref, r_ref):
    nrel = tr_ref.shape[0]
    rel = rel_ref[0]
    ioh = lax.broadcasted_iota(jnp.int32, (nrel, TB), 0)
    oh = (ioh == rel).astype(jnp.bfloat16)
    mg_ref[...] = lax.dot_general(
        oh, tr_ref[...], (((0,), (0,)), ((), ())),
        preferred_element_type=jnp.float32).astype(jnp.bfloat16)
    r_ref[...] = lax.dot_general(
        oh, re_ref[...], (((0,), (0,)), ((), ())),
        preferred_element_type=jnp.float32)


def _tca(rel3, tr_bf, re_bf, batch):
    nrel = tr_bf.shape[0]
    grid = (batch // TB,)
    return pl.pallas_call(
        _tca_body,
        grid=grid,
        in_specs=[
            pl.BlockSpec((1, 1, TB), lambda i: (i, 0, 0)),
            pl.BlockSpec((nrel, ENT_DIM * HID_DIM), lambda i: (0, 0)),
            pl.BlockSpec((nrel, HID_DIM), lambda i: (0, 0)),
        ],
        out_specs=[
            pl.BlockSpec((TB, ENT_DIM * HID_DIM), lambda i: (i, 0)),
            pl.BlockSpec((TB, HID_DIM), lambda i: (i, 0)),
        ],
        out_shape=[
            jax.ShapeDtypeStruct((batch, ENT_DIM * HID_DIM), jnp.bfloat16),
            jax.ShapeDtypeStruct((batch, HID_DIM), jnp.float32),
        ],
    )(rel3, tr_bf, re_bf)


def _tcb_body(h_ref, t_ref, mg_ref, r_ref, out_ref):
    dt_t = h_ref[...] - t_ref[...]
    i0 = lax.broadcasted_iota(jnp.int32, (ENT_DIM, ENT_DIM * HID_DIM), 0)
    i1 = lax.broadcasted_iota(jnp.int32, (ENT_DIM, ENT_DIM * HID_DIM), 1)
    tile_m = (i1 % ENT_DIM == i0).astype(jnp.bfloat16)
    dt = lax.dot_general(dt_t.astype(jnp.bfloat16), tile_m,
                         (((0,), (0,)), ((), ())),
                         preferred_element_type=jnp.float32)
    prod = mg_ref[...].astype(jnp.float32) * dt
    j0 = lax.broadcasted_iota(jnp.int32, (ENT_DIM * HID_DIM, HID_DIM), 0)
    j1 = lax.broadcasted_iota(jnp.int32, (ENT_DIM * HID_DIM, HID_DIM), 1)
    seg_m = (j0 // ENT_DIM == j1).astype(jnp.float32)
    p = jnp.dot(prod, seg_m, preferred_element_type=jnp.float32)
    s = p + r_ref[...]
    out_ref[...] = jnp.sqrt(jnp.sum(s * s, axis=1)).reshape(1, 1, TBB)


def _tcb(h_t, t_t, mg, r_g):
    batch = h_t.shape[1]
    grid = (batch // TBB,)
    out = pl.pallas_call(
        _tcb_body,
        grid=grid,
        in_specs=[
            pl.BlockSpec((ENT_DIM, TBB), lambda i: (0, i)),
            pl.BlockSpec((ENT_DIM, TBB), lambda i: (0, i)),
            pl.BlockSpec((TBB, ENT_DIM * HID_DIM), lambda i: (i, 0)),
            pl.BlockSpec((TBB, HID_DIM), lambda i: (i, 0)),
        ],
        out_specs=pl.BlockSpec((1, 1, TBB), lambda i: (i, 0, 0)),
        out_shape=jax.ShapeDtypeStruct((batch // TBB, 1, TBB), jnp.float32),
    )(h_t, t_t, mg, r_g)
    return out.reshape(batch)


def kernel(ent_emb, rel_emb, transfer, head_idx, tail_idx, rel_idx):
    batch = head_idx.shape[0]
    nch = batch // (NS * CHUNK)
    hidx3 = head_idx.astype(jnp.int32).reshape(NS, nch, CHUNK)
    tidx3 = tail_idx.astype(jnp.int32).reshape(NS, nch, CHUNK)
    h_t, t_t = _sc_gather_t(ent_emb.T, hidx3, tidx3, ent_emb.shape[0])

    rel3 = rel_idx.astype(jnp.int32).reshape(batch // TB, 1, TB)
    mg, r_g = _tca(rel3, transfer.astype(jnp.bfloat16),
                   rel_emb.astype(jnp.bfloat16), batch)
    return _tcb(h_t, t_t, mg, r_g)


# speedup vs baseline: 1.0071x; 1.0071x over previous
import functools

import jax
import jax.numpy as jnp
from jax import lax
from jax.experimental import pallas as pl
from jax.experimental.pallas import tpu as pltpu
from jax.experimental.pallas import tpu_sc as plsc

ENT_DIM = 64
HID_DIM = 32
NREL_PAD = 1024
NC, NS = 2, 16
CHUNK = 128
TB = 256
TBB = 512
FPC = ENT_DIM // NC


def _sc_gather_t(ent_t, hidx3, tidx3, n_ent):
    nch = hidx3.shape[1]
    bps = nch * CHUNK
    batch = NS * bps
    mesh = plsc.VectorSubcoreMesh(
        core_axis_name="c", subcore_axis_name="s",
        num_cores=NC, num_subcores=NS)

    @functools.partial(
        pl.kernel,
        out_type=[
            jax.ShapeDtypeStruct((ENT_DIM, batch), jnp.float32),
            jax.ShapeDtypeStruct((ENT_DIM, batch), jnp.float32),
        ],
        mesh=mesh,
        scratch_types=[
            pltpu.VMEM((nch, CHUNK), jnp.int32),
            pltpu.VMEM((nch, CHUNK), jnp.int32),
            pltpu.VMEM((2, bps), jnp.float32),
            pltpu.VMEM((bps,), jnp.float32),
            pltpu.VMEM_SHARED((n_ent,), jnp.float32),
            pltpu.VMEM_SHARED((n_ent,), jnp.float32),
            pltpu.SemaphoreType.DMA,
            pltpu.SemaphoreType.DMA,
            pltpu.SemaphoreType.DMA,
        ],
    )
    def k(ent_hbm, hidx_hbm, tidx_hbm, h_out, t_out,
          hv, tv, hbuf, tbuf, row_a, row_b, rsem, gsem, osem):
        cid = lax.axis_index("c")
        sid = lax.axis_index("s")
        f0 = cid * FPC
        pltpu.sync_copy(hidx_hbm.at[sid], hv)
        pltpu.sync_copy(tidx_hbm.at[sid], tv)
        bufs = (row_a, row_b)
        base = sid * bps

        def _stage_row(f, dst):
            pltpu.async_copy(ent_hbm.at[f0 + f], dst, rsem)

        def _drain_row(f, dst):
            pltpu.make_async_copy(ent_hbm.at[f0 + f], dst, rsem).wait()

        @pl.when(sid == 0)
        def _():
            _stage_row(0, bufs[0])

        for f in range(FPC):
            cur = bufs[f % 2]

            @pl.when(sid == 0)
            def _():
                _drain_row(f, cur)

            plsc.subcore_barrier()
            if f + 1 < FPC:
                @pl.when(sid == 0)
                def _():
                    _stage_row(f + 1, bufs[(f + 1) % 2])
            pb = f % 2
            if f >= 2:
                pltpu.make_async_copy(
                    hbuf.at[pb], h_out.at[f0 + f - 2, pl.ds(base, bps)],
                    osem).wait()
            if f >= 1:
                pltpu.make_async_copy(
                    tbuf, t_out.at[f0 + f - 1, pl.ds(base, bps)],
                    osem).wait()
            copies = []
            for j in range(nch):
                copies.append(pltpu.async_copy(
                    cur.at[hv.at[j]], hbuf.at[pb, pl.ds(j * CHUNK, CHUNK)],
                    gsem))
                copies.append(pltpu.async_copy(
                    cur.at[tv.at[j]], tbuf.at[pl.ds(j * CHUNK, CHUNK)],
                    gsem))
            for c in copies:
                c.wait()
            pltpu.async_copy(
                hbuf.at[pb], h_out.at[f0 + f, pl.ds(base, bps)], osem)
            pltpu.async_copy(
                tbuf, t_out.at[f0 + f, pl.ds(base, bps)], osem)
        pltpu.make_async_copy(
            hbuf.at[(FPC - 2) % 2],
            h_out.at[f0 + FPC - 2, pl.ds(base, bps)], osem).wait()
        pltpu.make_async_copy(
            hbuf.at[(FPC - 1) % 2],
            h_out.at[f0 + FPC - 1, pl.ds(base, bps)], osem).wait()
        pltpu.make_async_copy(
            tbuf, t_out.at[f0 + FPC - 1, pl.ds(base, bps)], osem).wait()

    return k(ent_t, hidx3, tidx3)


def _tca_body(rel_ref, tr_ref, re_ref, mg_ref, r_ref):
    nrel = tr_ref.shape[0]
    rel = rel_ref[0]
    ioh = lax.broadcasted_iota(jnp.int32, (nrel, TB), 0)
    oh = (ioh == rel).astype(jnp.bfloat16)
    mg_ref[...] = lax.dot_general(
        oh, tr_ref[...], (((0,), (0,)), ((), ())),
        preferred_element_type=jnp.float32).astype(jnp.bfloat16)
    r_ref[...] = lax.dot_general(
        oh, re_ref[...], (((0,), (0,)), ((), ())),
        preferred_element_type=jnp.float32)


def _tca(rel3, tr_bf, re_bf, batch):
    nrel = tr_bf.shape[0]
    grid = (batch // TB,)
    return pl.pallas_call(
        _tca_body,
        grid=grid,
        in_specs=[
            pl.BlockSpec((1, 1, TB), lambda i: (i, 0, 0)),
            pl.BlockSpec((nrel, ENT_DIM * HID_DIM), lambda i: (0, 0)),
            pl.BlockSpec((nrel, HID_DIM), lambda i: (0, 0)),
        ],
        out_specs=[
            pl.BlockSpec((TB, ENT_DIM * HID_DIM), lambda i: (i, 0)),
            pl.BlockSpec((TB, HID_DIM), lambda i: (i, 0)),
        ],
        out_shape=[
            jax.ShapeDtypeStruct((batch, ENT_DIM * HID_DIM), jnp.bfloat16),
            jax.ShapeDtypeStruct((batch, HID_DIM), jnp.float32),
        ],
    )(rel3, tr_bf, re_bf)


def _tcb_body(h_ref, t_ref, mg_ref, r_ref, out_ref):
    dt_t = h_ref[...] - t_ref[...]
    i0 = lax.broadcasted_iota(jnp.int32, (ENT_DIM, ENT_DIM * HID_DIM), 0)
    i1 = lax.broadcasted_iota(jnp.int32, (ENT_DIM, ENT_DIM * HID_DIM), 1)
    tile_m = (i1 % ENT_DIM == i0).astype(jnp.bfloat16)
    dt = lax.dot_general(dt_t.astype(jnp.bfloat16), tile_m,
                         (((0,), (0,)), ((), ())),
                         preferred_element_type=jnp.float32)
    prod = mg_ref[...].astype(jnp.float32) * dt
    j0 = lax.broadcasted_iota(jnp.int32, (ENT_DIM * HID_DIM, HID_DIM), 0)
    j1 = lax.broadcasted_iota(jnp.int32, (ENT_DIM * HID_DIM, HID_DIM), 1)
    seg_m = (j0 // ENT_DIM == j1).astype(jnp.float32)
    p = jnp.dot(prod, seg_m, preferred_element_type=jnp.float32)
    s = p + r_ref[...]
    out_ref[...] = jnp.sqrt(jnp.sum(s * s, axis=1)).reshape(1, 1, TBB)


def _tcb(h_t, t_t, mg, r_g):
    batch = h_t.shape[1]
    grid = (batch // TBB,)
    out = pl.pallas_call(
        _tcb_body,
        grid=grid,
        in_specs=[
            pl.BlockSpec((ENT_DIM, TBB), lambda i: (0, i)),
            pl.BlockSpec((ENT_DIM, TBB), lambda i: (0, i)),
            pl.BlockSpec((TBB, ENT_DIM * HID_DIM), lambda i: (i, 0)),
            pl.BlockSpec((TBB, HID_DIM), lambda i: (i, 0)),
        ],
        out_specs=pl.BlockSpec((1, 1, TBB), lambda i: (i, 0, 0)),
        out_shape=jax.ShapeDtypeStruct((batch // TBB, 1, TBB), jnp.float32),
    )(h_t, t_t, mg, r_g)
    return out.reshape(batch)


def kernel(ent_emb, rel_emb, transfer, head_idx, tail_idx, rel_idx):
    batch = head_idx.shape[0]
    nch = batch // (NS * CHUNK)
    hidx3 = head_idx.astype(jnp.int32).reshape(NS, nch, CHUNK)
    tidx3 = tail_idx.astype(jnp.int32).reshape(NS, nch, CHUNK)
    h_t, t_t = _sc_gather_t(ent_emb.T, hidx3, tidx3, ent_emb.shape[0])

    rel3 = rel_idx.astype(jnp.int32).reshape(batch // TB, 1, TB)
    mg, r_g = _tca(rel3, transfer.astype(jnp.bfloat16),
                   rel_emb.astype(jnp.bfloat16), batch)
    return _tcb(h_t, t_t, mg, r_g)


# speedup vs baseline: 1.0081x; 1.0009x over previous
import functools

import jax
import jax.numpy as jnp
from jax import lax
from jax.experimental import pallas as pl
from jax.experimental.pallas import tpu as pltpu
from jax.experimental.pallas import tpu_sc as plsc

ENT_DIM = 64
HID_DIM = 32
NREL_PAD = 1024
NC, NS = 2, 16
CHUNK = 128
TB = 256
TBB = 512
FPC = ENT_DIM // NC


def _sc_gather_t(ent_t, hidx3, tidx3, n_ent):
    nch = hidx3.shape[1]
    bps = nch * CHUNK
    batch = NS * bps
    mesh = plsc.VectorSubcoreMesh(
        core_axis_name="c", subcore_axis_name="s",
        num_cores=NC, num_subcores=NS)

    @functools.partial(
        pl.kernel,
        out_type=[
            jax.ShapeDtypeStruct((ENT_DIM, batch), jnp.float32),
            jax.ShapeDtypeStruct((ENT_DIM, batch), jnp.float32),
        ],
        mesh=mesh,
        scratch_types=[
            pltpu.VMEM((nch, CHUNK), jnp.int32),
            pltpu.VMEM((nch, CHUNK), jnp.int32),
            pltpu.VMEM((bps,), jnp.float32),
            pltpu.VMEM((bps,), jnp.float32),
            pltpu.VMEM_SHARED((n_ent,), jnp.float32),
            pltpu.VMEM_SHARED((n_ent,), jnp.float32),
            pltpu.SemaphoreType.DMA,
            pltpu.SemaphoreType.DMA,
        ],
    )
    def k(ent_hbm, hidx_hbm, tidx_hbm, h_out, t_out,
          hv, tv, hbuf, tbuf, row_a, row_b, rsem, gsem):
        cid = lax.axis_index("c")
        sid = lax.axis_index("s")
        f0 = cid * FPC
        pltpu.sync_copy(hidx_hbm.at[sid], hv)
        pltpu.sync_copy(tidx_hbm.at[sid], tv)
        bufs = (row_a, row_b)
        base = sid * bps

        def _stage_row(f, dst):
            pltpu.async_copy(ent_hbm.at[f0 + f], dst, rsem)

        def _drain_row(f, dst):
            pltpu.make_async_copy(ent_hbm.at[f0 + f], dst, rsem).wait()

        @pl.when(sid == 0)
        def _():
            _stage_row(0, bufs[0])

        for f in range(FPC):
            cur = bufs[f % 2]

            @pl.when(sid == 0)
            def _():
                _drain_row(f, cur)

            plsc.subcore_barrier()
            if f + 1 < FPC:
                @pl.when(sid == 0)
                def _():
                    _stage_row(f + 1, bufs[(f + 1) % 2])
            copies = []
            for j in range(nch):
                copies.append(pltpu.async_copy(
                    cur.at[hv.at[j]], hbuf.at[pl.ds(j * CHUNK, CHUNK)],
                    gsem))
                copies.append(pltpu.async_copy(
                    cur.at[tv.at[j]], tbuf.at[pl.ds(j * CHUNK, CHUNK)],
                    gsem))
            for c in copies:
                c.wait()
            pltpu.sync_copy(hbuf, h_out.at[f0 + f, pl.ds(base, bps)])
            pltpu.sync_copy(tbuf, t_out.at[f0 + f, pl.ds(base, bps)])

    return k(ent_t, hidx3, tidx3)


def _tca_body(rel_ref, tr_ref, re_ref, mg_ref, r_ref):
    nrel = tr_ref.shape[0]
    rel = rel_ref[0]
    ioh = lax.broadcasted_iota(jnp.int32, (nrel, TB), 0)
    oh = (ioh == rel).astype(jnp.bfloat16)
    mg_ref[...] = lax.dot_general(
        oh, tr_ref[...], (((0,), (0,)), ((), ())),
        preferred_element_type=jnp.float32).astype(jnp.bfloat16)
    r_ref[...] = lax.dot_general(
        oh, re_ref[...], (((0,), (0,)), ((), ())),
        preferred_element_type=jnp.float32)


def _tca(rel3, tr_bf, re_bf, batch):
    nrel = tr_bf.shape[0]
    grid = (batch // TB,)
    return pl.pallas_call(
        _tca_body,
        grid=grid,
        in_specs=[
            pl.BlockSpec((1, 1, TB), lambda i: (i, 0, 0)),
            pl.BlockSpec((nrel, ENT_DIM * HID_DIM), lambda i: (0, 0)),
            pl.BlockSpec((nrel, HID_DIM), lambda i: (0, 0)),
        ],
        out_specs=[
            pl.BlockSpec((TB, ENT_DIM * HID_DIM), lambda i: (i, 0)),
            pl.BlockSpec((TB, HID_DIM), lambda i: (i, 0)),
        ],
        out_shape=[
            jax.ShapeDtypeStruct((batch, ENT_DIM * HID_DIM), jnp.bfloat16),
            jax.ShapeDtypeStruct((batch, HID_DIM), jnp.float32),
        ],
    )(rel3, tr_bf, re_bf)


def _tcb_body(h_ref, t_ref, mg_ref, r_ref, out_ref):
    dt_t = h_ref[...] - t_ref[...]
    i0 = lax.broadcasted_iota(jnp.int32, (ENT_DIM, ENT_DIM * HID_DIM), 0)
    i1 = lax.broadcasted_iota(jnp.int32, (ENT_DIM, ENT_DIM * HID_DIM), 1)
    tile_m = (i1 % ENT_DIM == i0).astype(jnp.bfloat16)
    dt = lax.dot_general(dt_t.astype(jnp.bfloat16), tile_m,
                         (((0,), (0,)), ((), ())),
                         preferred_element_type=jnp.float32)
    prod = mg_ref[...].astype(jnp.float32) * dt
    j0 = lax.broadcasted_iota(jnp.int32, (ENT_DIM * HID_DIM, HID_DIM), 0)
    j1 = lax.broadcasted_iota(jnp.int32, (ENT_DIM * HID_DIM, HID_DIM), 1)
    seg_m = (j0 // ENT_DIM == j1).astype(jnp.float32)
    p = jnp.dot(prod, seg_m, preferred_element_type=jnp.float32)
    s = p + r_ref[...]
    out_ref[...] = jnp.sqrt(jnp.sum(s * s, axis=1)).reshape(1, 1, TBB)


def _tcb(h_t, t_t, mg, r_g):
    batch = h_t.shape[1]
    grid = (batch // TBB,)
    out = pl.pallas_call(
        _tcb_body,
        grid=grid,
        in_specs=[
            pl.BlockSpec((ENT_DIM, TBB), lambda i: (0, i)),
            pl.BlockSpec((ENT_DIM, TBB), lambda i: (0, i)),
            pl.BlockSpec((TBB, ENT_DIM * HID_DIM), lambda i: (i, 0)),
            pl.BlockSpec((TBB, HID_DIM), lambda i: (i, 0)),
        ],
        out_specs=pl.BlockSpec((1, 1, TBB), lambda i: (i, 0, 0)),
        out_shape=jax.ShapeDtypeStruct((batch // TBB, 1, TBB), jnp.float32),
    )(h_t, t_t, mg, r_g)
    return out.reshape(batch)


def kernel(ent_emb, rel_emb, transfer, head_idx, tail_idx, rel_idx):
    batch = head_idx.shape[0]
    nch = batch // (NS * CHUNK)
    hidx3 = head_idx.astype(jnp.int32).reshape(NS, nch, CHUNK)
    tidx3 = tail_idx.astype(jnp.int32).reshape(NS, nch, CHUNK)
    h_t, t_t = _sc_gather_t(ent_emb.T, hidx3, tidx3, ent_emb.shape[0])

    rel3 = rel_idx.astype(jnp.int32).reshape(batch // TB, 1, TB)
    mg, r_g = _tca(rel3, transfer.astype(jnp.bfloat16),
                   rel_emb.astype(jnp.bfloat16), batch)
    return _tcb(h_t, t_t, mg, r_g)


# speedup vs baseline: 1.0135x; 1.0054x over previous
import functools

import jax
import jax.numpy as jnp
from jax import lax
from jax.experimental import pallas as pl
from jax.experimental.pallas import tpu as pltpu
from jax.experimental.pallas import tpu_sc as plsc

ENT_DIM = 64
HID_DIM = 32
NREL_PAD = 1024
NC, NS = 2, 16
CHUNK = 128
TB = 256
TBB = 512
FPC = ENT_DIM // NC


def _sc_gather_t(ent_t, hidx3, tidx3, n_ent):
    nch = hidx3.shape[1]
    bps = nch * CHUNK
    batch = NS * bps
    mesh = plsc.VectorSubcoreMesh(
        core_axis_name="c", subcore_axis_name="s",
        num_cores=NC, num_subcores=NS)

    @functools.partial(
        pl.kernel,
        out_type=[
            jax.ShapeDtypeStruct((ENT_DIM, batch), jnp.float32),
            jax.ShapeDtypeStruct((ENT_DIM, batch), jnp.float32),
        ],
        mesh=mesh,
        scratch_types=[
            pltpu.VMEM((nch, CHUNK), jnp.int32),
            pltpu.VMEM((nch, CHUNK), jnp.int32),
            pltpu.VMEM((bps,), jnp.float32),
            pltpu.VMEM((bps,), jnp.float32),
            pltpu.VMEM_SHARED((n_ent,), jnp.float32),
            pltpu.VMEM_SHARED((n_ent,), jnp.float32),
            pltpu.SemaphoreType.DMA,
            pltpu.SemaphoreType.DMA,
        ],
    )
    def k(ent_hbm, hidx_hbm, tidx_hbm, h_out, t_out,
          hv, tv, hbuf, tbuf, row_a, row_b, rsem, gsem):
        cid = lax.axis_index("c")
        sid = lax.axis_index("s")
        f0 = cid * FPC
        pltpu.sync_copy(hidx_hbm.at[sid], hv)
        pltpu.sync_copy(tidx_hbm.at[sid], tv)
        bufs = (row_a, row_b)
        base = sid * bps

        def _stage_row(f, dst):
            pltpu.async_copy(ent_hbm.at[f0 + f], dst, rsem)

        def _drain_row(f, dst):
            pltpu.make_async_copy(ent_hbm.at[f0 + f], dst, rsem).wait()

        @pl.when(sid == 0)
        def _():
            _stage_row(0, bufs[0])

        for f in range(FPC):
            cur = bufs[f % 2]

            @pl.when(sid == 0)
            def _():
                _drain_row(f, cur)

            plsc.subcore_barrier()
            if f + 1 < FPC:
                @pl.when(sid == 0)
                def _():
                    _stage_row(f + 1, bufs[(f + 1) % 2])
            copies = []
            for j in range(nch):
                copies.append(pltpu.async_copy(
                    cur.at[hv.at[j]], hbuf.at[pl.ds(j * CHUNK, CHUNK)],
                    gsem))
                copies.append(pltpu.async_copy(
                    cur.at[tv.at[j]], tbuf.at[pl.ds(j * CHUNK, CHUNK)],
                    gsem))
            for c in copies:
                c.wait()
            pltpu.sync_copy(hbuf, h_out.at[f0 + f, pl.ds(base, bps)])
            pltpu.sync_copy(tbuf, t_out.at[f0 + f, pl.ds(base, bps)])

    return k(ent_t, hidx3, tidx3)


def _tca_body(rel_ref, tr_ref, re_ref, mg_ref, r_ref, trb_ref, reb_ref):
    nrel = tr_ref.shape[0]

    @pl.when(pl.program_id(0) == 0)
    def _():
        trb_ref[...] = tr_ref[...].astype(jnp.bfloat16)
        reb_ref[...] = re_ref[...].astype(jnp.bfloat16)

    rel = rel_ref[0]
    ioh = lax.broadcasted_iota(jnp.int32, (nrel, TB), 0)
    oh = (ioh == rel).astype(jnp.bfloat16)
    mg_ref[...] = lax.dot_general(
        oh, trb_ref[...], (((0,), (0,)), ((), ())),
        preferred_element_type=jnp.float32).astype(jnp.bfloat16)
    r_ref[...] = lax.dot_general(
        oh, reb_ref[...], (((0,), (0,)), ((), ())),
        preferred_element_type=jnp.float32)


def _tca(rel3, tr_bf, re_bf, batch):
    nrel = tr_bf.shape[0]
    grid = (batch // TB,)
    return pl.pallas_call(
        _tca_body,
        grid=grid,
        in_specs=[
            pl.BlockSpec((1, 1, TB), lambda i: (i, 0, 0)),
            pl.BlockSpec((nrel, ENT_DIM * HID_DIM), lambda i: (0, 0)),
            pl.BlockSpec((nrel, HID_DIM), lambda i: (0, 0)),
        ],
        out_specs=[
            pl.BlockSpec((TB, ENT_DIM * HID_DIM), lambda i: (i, 0)),
            pl.BlockSpec((TB, HID_DIM), lambda i: (i, 0)),
        ],
        out_shape=[
            jax.ShapeDtypeStruct((batch, ENT_DIM * HID_DIM), jnp.bfloat16),
            jax.ShapeDtypeStruct((batch, HID_DIM), jnp.float32),
        ],
        scratch_shapes=[
            pltpu.VMEM((nrel, ENT_DIM * HID_DIM), jnp.bfloat16),
            pltpu.VMEM((nrel, HID_DIM), jnp.bfloat16),
        ],
    )(rel3, tr_bf, re_bf)


def _tcb_body(h_ref, t_ref, mg_ref, r_ref, out_ref):
    dt_t = h_ref[...] - t_ref[...]
    i0 = lax.broadcasted_iota(jnp.int32, (ENT_DIM, ENT_DIM * HID_DIM), 0)
    i1 = lax.broadcasted_iota(jnp.int32, (ENT_DIM, ENT_DIM * HID_DIM), 1)
    tile_m = (i1 % ENT_DIM == i0).astype(jnp.bfloat16)
    dt = lax.dot_general(dt_t.astype(jnp.bfloat16), tile_m,
                         (((0,), (0,)), ((), ())),
                         preferred_element_type=jnp.float32)
    prod = mg_ref[...].astype(jnp.float32) * dt
    j0 = lax.broadcasted_iota(jnp.int32, (ENT_DIM * HID_DIM, HID_DIM), 0)
    j1 = lax.broadcasted_iota(jnp.int32, (ENT_DIM * HID_DIM, HID_DIM), 1)
    seg_m = (j0 // ENT_DIM == j1).astype(jnp.float32)
    p = jnp.dot(prod, seg_m, preferred_element_type=jnp.float32)
    s = p + r_ref[...]
    out_ref[...] = jnp.sqrt(jnp.sum(s * s, axis=1)).reshape(1, 1, TBB)


def _tcb(h_t, t_t, mg, r_g):
    batch = h_t.shape[1]
    grid = (batch // TBB,)
    out = pl.pallas_call(
        _tcb_body,
        grid=grid,
        in_specs=[
            pl.BlockSpec((ENT_DIM, TBB), lambda i: (0, i)),
            pl.BlockSpec((ENT_DIM, TBB), lambda i: (0, i)),
            pl.BlockSpec((TBB, ENT_DIM * HID_DIM), lambda i: (i, 0)),
            pl.BlockSpec((TBB, HID_DIM), lambda i: (i, 0)),
        ],
        out_specs=pl.BlockSpec((1, 1, TBB), lambda i: (i, 0, 0)),
        out_shape=jax.ShapeDtypeStruct((batch // TBB, 1, TBB), jnp.float32),
    )(h_t, t_t, mg, r_g)
    return out.reshape(batch)


def kernel(ent_emb, rel_emb, transfer, head_idx, tail_idx, rel_idx):
    batch = head_idx.shape[0]
    nch = batch // (NS * CHUNK)
    hidx3 = head_idx.astype(jnp.int32).reshape(NS, nch, CHUNK)
    tidx3 = tail_idx.astype(jnp.int32).reshape(NS, nch, CHUNK)
    h_t, t_t = _sc_gather_t(ent_emb.T, hidx3, tidx3, ent_emb.shape[0])

    rel3 = rel_idx.astype(jnp.int32).reshape(batch // TB, 1, TB)
    mg, r_g = _tca(rel3, transfer, rel_emb, batch)
    return _tcb(h_t, t_t, mg, r_g)
